# Initial kernel scaffold; baseline (speedup 1.0000x reference)
import jax
import jax.numpy as jnp
from jax.experimental import pallas as pl


def kernel(x, efeat, degs, norm, params, edge_index):
    raise NotImplementedError("write your pallas kernel here")


# speedup vs baseline: 1.8461x; 1.8461x over previous
import functools

import jax
import jax.numpy as jnp
from jax import lax
from jax.experimental import pallas as pl
from jax.experimental.pallas import tpu as pltpu
from jax.experimental.pallas import tpu_sc as plsc

N = 10000
D = 128
E = 320000
ED = 7

NC = 2
NS = 16
NW = NC * NS
CH = 128
CPS = 4
SUP_E = CPS * CH
NSUP = 20
EPT = NSUP * SUP_E
E_PAD = EPT * NW
ROWS_PT = 624
TAIL = N - NS * ROWS_PT
NCH = D // 16


_RB = 1000


def _pre_body(x_ref, w_ref, wr_ref, bias_ref, degs_ref,
              table_ref, self_ref, res_ref):
    x = x_ref[...]
    nf = jnp.dot(x, w_ref[...], preferred_element_type=jnp.float32) + bias_ref[0]
    table_ref[...] = nf + bias_ref[1]
    self_ref[...] = jnp.maximum(nf + bias_ref[3], 0.0) / degs_ref[...]
    res_ref[...] = jnp.maximum(
        jnp.dot(x, wr_ref[...], preferred_element_type=jnp.float32) + bias_ref[2],
        0.0)


def _pre_call(x, W, Wr, bias, degs):
    return pl.pallas_call(
        _pre_body,
        grid=(N // _RB,),
        in_specs=[
            pl.BlockSpec((_RB, D), lambda i: (i, 0)),
            pl.BlockSpec((D, D), lambda i: (0, 0)),
            pl.BlockSpec((D, D), lambda i: (0, 0)),
            pl.BlockSpec((8, D), lambda i: (0, 0)),
            pl.BlockSpec((_RB, 1), lambda i: (i, 0)),
        ],
        out_specs=[
            pl.BlockSpec((_RB, D), lambda i: (i, 0)),
            pl.BlockSpec((_RB, D), lambda i: (i, 0)),
            pl.BlockSpec((_RB, D), lambda i: (i, 0)),
        ],
        out_shape=[
            jax.ShapeDtypeStruct((N, D), jnp.float32),
            jax.ShapeDtypeStruct((N, D), jnp.float32),
            jax.ShapeDtypeStruct((N, D), jnp.float32),
        ],
    )(x, W, Wr, bias, degs)


def _mp_body(table_h, src_h, dst_h, ne_h, w_h, z_h, out_h,
             w_v, src_v, dst_v, ne_v, g0, g1, acc,
             gsem0, gsem1, ssem0, ssem1):
    cid = lax.axis_index("c")
    sid = lax.axis_index("s")
    wid = cid * NS + sid

    pltpu.sync_copy(w_h, w_v)
    pltpu.sync_copy(z_h.at[pl.ds(0, ROWS_PT)],
                    acc.at[pl.ds(sid * ROWS_PT, ROWS_PT)])

    @pl.when(sid == NS - 1)
    def _zero_tail():
        pltpu.sync_copy(z_h.at[pl.ds(0, TAIL)],
                        acc.at[pl.ds(NS * ROWS_PT, TAIL)])

    plsc.subcore_barrier()

    wv = [[w_v[k, pl.ds(c * 16, 16)] for c in range(NCH)] for k in range(ED)]

    gbufs = (g0, g1)
    gsems = (gsem0, gsem1)
    ssems = (ssem0, ssem1)

    def superchunk(s, carry):
        pltpu.sync_copy(src_h.at[wid, s], src_v)
        pltpu.sync_copy(dst_h.at[wid, s], dst_v)
        pltpu.sync_copy(ne_h.at[wid, s], ne_v)
        pltpu.async_copy(table_h.at[src_v.at[0]], g0, gsem0)
        for j in range(CPS):
            b = j % 2
            gbuf = gbufs[b]
            pltpu.make_async_copy(table_h.at[src_v.at[j]], gbuf, gsems[b]).wait()
            if j + 1 < CPS:
                ob = 1 - b
                if j >= 1:
                    pltpu.make_async_copy(
                        gbufs[ob], acc.at[dst_v.at[j - 1]], ssems[ob]).wait()
                pltpu.async_copy(table_h.at[src_v.at[j + 1]], gbufs[ob], gsems[ob])

            def edge_body(i, _, j=j, gbuf=gbuf):
                e = j * CH + i
                nev = ne_v[e, :]
                f = [nev[k] for k in range(ED)]
                nrm = nev[ED]
                for c in range(NCH):
                    v = gbuf[i, pl.ds(c * 16, 16)]
                    for k in range(ED):
                        v = v + f[k] * wv[k][c]
                    gbuf[i, pl.ds(c * 16, 16)] = jnp.maximum(v, 0.0) * nrm
                return 0

            lax.fori_loop(0, CH, edge_body, 0)
            pltpu.async_copy(gbuf, acc.at[dst_v.at[j]], ssems[b], add=True)
        pltpu.make_async_copy(g0, acc.at[dst_v.at[CPS - 2]], ssem0).wait()
        pltpu.make_async_copy(g1, acc.at[dst_v.at[CPS - 1]], ssem1).wait()
        return carry

    lax.fori_loop(0, NSUP, superchunk, 0)

    plsc.subcore_barrier()
    pltpu.sync_copy(acc.at[pl.ds(sid * ROWS_PT, ROWS_PT)],
                    out_h.at[cid].at[pl.ds(sid * ROWS_PT, ROWS_PT)])

    @pl.when(sid == NS - 1)
    def _copy_tail():
        pltpu.sync_copy(acc.at[pl.ds(NS * ROWS_PT, TAIL)],
                        out_h.at[cid].at[pl.ds(NS * ROWS_PT, TAIL)])


_mp_call = pl.kernel(
    _mp_body,
    out_type=jax.ShapeDtypeStruct((NC, N, D), jnp.float32),
    mesh=plsc.VectorSubcoreMesh(core_axis_name="c", subcore_axis_name="s"),
    compiler_params=pltpu.CompilerParams(use_tc_tiling_on_sc=False),
    scratch_types=[
        pltpu.VMEM((8, D), jnp.float32),
        pltpu.VMEM((CPS, CH), jnp.int32),
        pltpu.VMEM((CPS, CH), jnp.int32),
        pltpu.VMEM((SUP_E, 16), jnp.float32),
        pltpu.VMEM((CH, D), jnp.float32),
        pltpu.VMEM((CH, D), jnp.float32),
        pltpu.VMEM_SHARED((N, D), jnp.float32),
        pltpu.SemaphoreType.DMA,
        pltpu.SemaphoreType.DMA,
        pltpu.SemaphoreType.DMA,
        pltpu.SemaphoreType.DMA,
    ],
)


def _post_body(ft2_ref, self_ref, res_ref, gb_ref, y_ref):
    ft = ft2_ref[0] + ft2_ref[1] + self_ref[...]
    y = jnp.maximum(ft, 0.0) + res_ref[...]
    mean = jnp.mean(y, axis=0, keepdims=True)
    var = jnp.mean((y - mean) * (y - mean), axis=0, keepdims=True)
    y_ref[...] = (y - mean) / jnp.sqrt(var + 1e-5) * gb_ref[0] + gb_ref[1]


def _post_call(ft2, selfterm, res, gb):
    return pl.pallas_call(
        _post_body,
        out_shape=jax.ShapeDtypeStruct((N, D), jnp.float32),
    )(ft2, selfterm, res, gb)


def kernel(x, efeat, degs, norm, params, edge_index):
    src = edge_index[0]
    dst = edge_index[1]
    pad = E_PAD - E
    src_p = jnp.concatenate(
        [src, jnp.zeros((pad,), jnp.int32)]).reshape(NW, NSUP, CPS, CH)
    dst_p = jnp.concatenate(
        [dst, jnp.zeros((pad,), jnp.int32)]).reshape(NW, NSUP, CPS, CH)
    ne = jnp.concatenate(
        [efeat, norm, jnp.zeros((E, 8), jnp.float32)], axis=1)
    ne_p = jnp.concatenate(
        [ne, jnp.zeros((pad, 16), jnp.float32)], axis=0).reshape(
            NW, NSUP, SUP_E, 16)
    zeros_t = jnp.zeros((ROWS_PT, D), jnp.float32)

    def layer(h, p):
        bias = jnp.zeros((8, D), jnp.float32)
        bias = bias.at[0].set(p['b']).at[1].set(p['be'])
        bias = bias.at[2].set(p['br']).at[3].set(p['root'][0])
        table, selfterm, res = _pre_call(h, p['W'], p['Wr'], bias, degs)
        wmat = jnp.zeros((8, D), jnp.float32).at[:ED].set(p['We'])
        ft2 = _mp_call(table, src_p, dst_p, ne_p, wmat, zeros_t)
        gb = jnp.zeros((8, D), jnp.float32)
        gb = gb.at[0].set(p['gamma']).at[1].set(p['beta'])
        return _post_call(ft2, selfterm, res, gb)

    h = layer(x, params['layer0'])
    h = layer(h, params['layer1'])
    return h


# speedup vs baseline: 2.3088x; 1.2507x over previous
import functools

import jax
import jax.numpy as jnp
import numpy as np
from jax import lax
from jax.experimental import pallas as pl
from jax.experimental.pallas import tpu as pltpu
from jax.experimental.pallas import tpu_sc as plsc

N = 10000
D = 128
E = 320000
ED = 7

NC = 2
NS = 16
NW = NC * NS
CH = 128
CPS = 8
SUP_E = CPS * CH
NSUP = 10
EPT = NSUP * SUP_E
E_PAD = EPT * NW
ROWS_PT = 624
TAIL = N - NS * ROWS_PT
NCH = D // 16


_RB = 1000


def _pre_body(x_ref, w_ref, wr_ref, bias_ref, degs_ref,
              table_ref, self_ref, res_ref):
    x = x_ref[...]
    nf = jnp.dot(x, w_ref[...], preferred_element_type=jnp.float32) + bias_ref[0]
    table_ref[...] = nf + bias_ref[1]
    self_ref[...] = jnp.maximum(nf + bias_ref[3], 0.0) / degs_ref[...]
    res_ref[...] = jnp.maximum(
        jnp.dot(x, wr_ref[...], preferred_element_type=jnp.float32) + bias_ref[2],
        0.0)


def _pre_call(x, W, Wr, bias, degs):
    return pl.pallas_call(
        _pre_body,
        grid=(N // _RB,),
        in_specs=[
            pl.BlockSpec((_RB, D), lambda i: (i, 0)),
            pl.BlockSpec((D, D), lambda i: (0, 0)),
            pl.BlockSpec((D, D), lambda i: (0, 0)),
            pl.BlockSpec((8, D), lambda i: (0, 0)),
            pl.BlockSpec((_RB, 1), lambda i: (i, 0)),
        ],
        out_specs=[
            pl.BlockSpec((_RB, D), lambda i: (i, 0)),
            pl.BlockSpec((_RB, D), lambda i: (i, 0)),
            pl.BlockSpec((_RB, D), lambda i: (i, 0)),
        ],
        out_shape=[
            jax.ShapeDtypeStruct((N, D), jnp.float32),
            jax.ShapeDtypeStruct((N, D), jnp.float32),
            jax.ShapeDtypeStruct((N, D), jnp.float32),
        ],
    )(x, W, Wr, bias, degs)


def _mp_body(table_h, src_h, dst_h, ne_h, w_h, z_h, out_h,
             w_v, src_v, dst_v, ne_v, g0, g1, s0, s1, acc,
             gsem0, gsem1, ssem0, ssem1):
    cid = lax.axis_index("c")
    sid = lax.axis_index("s")
    wid = cid * NS + sid

    pltpu.sync_copy(w_h, w_v)
    pltpu.sync_copy(z_h.at[pl.ds(0, ROWS_PT)],
                    acc.at[pl.ds(sid * ROWS_PT, ROWS_PT)])

    @pl.when(sid == NS - 1)
    def _zero_tail():
        pltpu.sync_copy(z_h.at[pl.ds(0, TAIL)],
                        acc.at[pl.ds(NS * ROWS_PT, TAIL)])

    plsc.subcore_barrier()

    wv = [[w_v[k, pl.ds(c * 16, 16)] for c in range(NCH)] for k in range(ED)]

    gbufs = (g0, g1)
    sbufs = (s0, s1)
    gsems = (gsem0, gsem1)
    ssems = (ssem0, ssem1)

    def superchunk(s, carry):
        pltpu.sync_copy(src_h.at[wid, s], src_v)
        pltpu.sync_copy(dst_h.at[wid, s], dst_v)
        pltpu.sync_copy(ne_h.at[wid, s], ne_v)
        pltpu.async_copy(table_h.at[src_v.at[0]], g0, gsem0)
        for j in range(CPS):
            b = j % 2
            gbuf = gbufs[b]
            sbuf = sbufs[b]
            pltpu.make_async_copy(table_h.at[src_v.at[j]], gbuf, gsems[b]).wait()
            if j + 1 < CPS:
                pltpu.async_copy(table_h.at[src_v.at[j + 1]],
                                 gbufs[1 - b], gsems[1 - b])
            if j >= 2:
                pltpu.make_async_copy(
                    sbuf, acc.at[dst_v.at[j - 2]], ssems[b]).wait()

            def edge_body(i, _, j=j, gbuf=gbuf, sbuf=sbuf):
                nev = ne_v[j * CH + i, :]
                f = [nev[k] for k in range(ED)]
                nrm = nev[ED]
                vs = []
                for c in range(NCH):
                    v = gbuf[i, pl.ds(c * 16, 16)]
                    for k in range(ED):
                        v = v + f[k] * wv[k][c]
                    vs.append(jnp.maximum(v, 0.0) * nrm)
                for c2 in range(NCH // 2):
                    packed = plsc.pack(vs[2 * c2], vs[2 * c2 + 1],
                                       format=plsc.PackFormat.INTERLEAVED)
                    sbuf[i, pl.ds(c2 * 32, 32)] = packed
                return 0

            lax.fori_loop(0, CH, edge_body, 0)
            pltpu.async_copy(sbuf, acc.at[dst_v.at[j]], ssems[b], add=True)
        pltpu.make_async_copy(s0, acc.at[dst_v.at[CPS - 2]], ssem0).wait()
        pltpu.make_async_copy(s1, acc.at[dst_v.at[CPS - 1]], ssem1).wait()
        return carry

    lax.fori_loop(0, NSUP, superchunk, 0)

    plsc.subcore_barrier()
    pltpu.sync_copy(acc.at[pl.ds(sid * ROWS_PT, ROWS_PT)],
                    out_h.at[cid].at[pl.ds(sid * ROWS_PT, ROWS_PT)])

    @pl.when(sid == NS - 1)
    def _copy_tail():
        pltpu.sync_copy(acc.at[pl.ds(NS * ROWS_PT, TAIL)],
                        out_h.at[cid].at[pl.ds(NS * ROWS_PT, TAIL)])


_mp_call = pl.kernel(
    _mp_body,
    out_type=jax.ShapeDtypeStruct((NC, N, D), jnp.bfloat16),
    mesh=plsc.VectorSubcoreMesh(core_axis_name="c", subcore_axis_name="s"),
    compiler_params=pltpu.CompilerParams(use_tc_tiling_on_sc=False,
                                         needs_layout_passes=False),
    scratch_types=[
        pltpu.VMEM((8, D), jnp.float32),
        pltpu.VMEM((CPS, CH), jnp.int32),
        pltpu.VMEM((CPS, CH), jnp.int32),
        pltpu.VMEM((SUP_E, 16), jnp.float32),
        pltpu.VMEM((CH, D), jnp.float32),
        pltpu.VMEM((CH, D), jnp.float32),
        pltpu.VMEM((CH, D), jnp.bfloat16),
        pltpu.VMEM((CH, D), jnp.bfloat16),
        pltpu.VMEM_SHARED((N, D), jnp.bfloat16),
        pltpu.SemaphoreType.DMA,
        pltpu.SemaphoreType.DMA,
        pltpu.SemaphoreType.DMA,
        pltpu.SemaphoreType.DMA,
    ],
)


def _post_body(ft2_ref, self_ref, res_ref, gb_ref, unperm_ref, y_ref):
    fts = (ft2_ref[0] + ft2_ref[1]).astype(jnp.float32)
    ft = jnp.dot(fts, unperm_ref[...], preferred_element_type=jnp.float32)
    y = jnp.maximum(ft + self_ref[...], 0.0) + res_ref[...]
    mean = jnp.mean(y, axis=0, keepdims=True)
    var = jnp.mean((y - mean) * (y - mean), axis=0, keepdims=True)
    y_ref[...] = (y - mean) / jnp.sqrt(var + 1e-5) * gb_ref[0] + gb_ref[1]


def _post_call(ft2, selfterm, res, gb, unperm):
    return pl.pallas_call(
        _post_body,
        out_shape=jax.ShapeDtypeStruct((N, D), jnp.float32),
    )(ft2, selfterm, res, gb, unperm)


def _build_unperm():
    s = np.arange(D)
    f = 32 * (s // 32) + 16 * (s % 2) + (s % 32) // 2
    m = np.zeros((D, D), np.float32)
    m[s, f] = 1.0
    return m


_UNPERM = _build_unperm()


def kernel(x, efeat, degs, norm, params, edge_index):
    src = edge_index[0]
    dst = edge_index[1]
    pad = E_PAD - E
    src_p = jnp.concatenate(
        [src, jnp.zeros((pad,), jnp.int32)]).reshape(NW, NSUP, CPS, CH)
    dst_p = jnp.concatenate(
        [dst, jnp.zeros((pad,), jnp.int32)]).reshape(NW, NSUP, CPS, CH)
    ne = jnp.concatenate(
        [efeat, norm, jnp.zeros((E, 8), jnp.float32)], axis=1)
    ne_p = jnp.concatenate(
        [ne, jnp.zeros((pad, 16), jnp.float32)], axis=0).reshape(
            NW, NSUP, SUP_E, 16)
    zeros_t = jnp.zeros((ROWS_PT, D), jnp.bfloat16)
    unperm = jnp.asarray(_UNPERM)

    def layer(h, p):
        bias = jnp.zeros((8, D), jnp.float32)
        bias = bias.at[0].set(p['b']).at[1].set(p['be'])
        bias = bias.at[2].set(p['br']).at[3].set(p['root'][0])
        table, selfterm, res = _pre_call(h, p['W'], p['Wr'], bias, degs)
        wmat = jnp.zeros((8, D), jnp.float32).at[:ED].set(p['We'])
        ft2 = _mp_call(table, src_p, dst_p, ne_p, wmat, zeros_t)
        gb = jnp.zeros((8, D), jnp.float32)
        gb = gb.at[0].set(p['gamma']).at[1].set(p['beta'])
        return _post_call(ft2, selfterm, res, gb, unperm)

    h = layer(x, params['layer0'])
    h = layer(h, params['layer1'])
    return h
